# Initial kernel scaffold; baseline (speedup 1.0000x reference)
import jax
import jax.numpy as jnp
from jax.experimental import pallas as pl


def kernel(input_encoded, memory_mean, memory_logvar):
    raise NotImplementedError("write your pallas kernel here")


# speedup vs baseline: 5.5887x; 5.5887x over previous
import math

import jax
import jax.numpy as jnp
from jax.experimental import pallas as pl

ALPHA = 0.1


def _compute_body(x_ref, mm_ref, z_ref, kl_ref, oh_ref, delta_ref):
    X = x_ref[...]
    MM = mm_ref[...]
    B, C = X.shape
    M = MM.shape[0]
    sims = jax.lax.dot_general(X, MM, (((1,), (1,)), ((), ())),
                               preferred_element_type=jnp.float32)
    closest = jnp.argmax(sims, axis=1)
    onehot = (jax.lax.broadcasted_iota(jnp.int32, (B, M), 1)
              == closest[:, None])
    oh_f = onehot.astype(jnp.float32)
    gathered = jax.lax.dot_general(oh_f, MM, (((1,), (0,)), ((), ())),
                                   preferred_element_type=jnp.float32)
    diff = X - gathered
    delta = ALPHA * diff
    xsq = jnp.sum(X * X, axis=1)
    s_at = jnp.sum(sims * oh_f, axis=1)
    corr = (1.0 - ALPHA) * s_at + ALPHA * xsq
    scores = jnp.where(onehot, corr[:, None], sims) * (1.0 / math.sqrt(C))
    smax = jnp.max(scores, axis=1, keepdims=True)
    e = jnp.exp(scores - smax)
    w = e / jnp.sum(e, axis=1, keepdims=True)
    z = jax.lax.dot_general(w, MM, (((1,), (0,)), ((), ())),
                            preferred_element_type=jnp.float32)
    w_at = jnp.sum(w * oh_f, axis=1)
    z = z + w_at[:, None] * delta
    z_ref[...] = z
    kl_ref[...] = 0.5 * (jnp.sum(diff * diff, axis=1)
                         + jnp.sum((z - X) ** 2, axis=1))
    oh_ref[...] = oh_f
    delta_ref[...] = delta


def _writer_body(mm_ref, oh_ref, delta_ref, post_ref):
    MM = mm_ref[...]
    oh = oh_ref[...]
    delta = delta_ref[...]
    post_ref[...] = MM[None, :, :] + oh[:, :, None] * delta[:, None, :]


def kernel(input_encoded, memory_mean, memory_logvar):
    del memory_logvar
    B, C = input_encoded.shape
    M = memory_mean.shape[0]

    z, kl, oh, delta = pl.pallas_call(
        _compute_body,
        out_shape=[
            jax.ShapeDtypeStruct((B, C), jnp.float32),
            jax.ShapeDtypeStruct((B,), jnp.float32),
            jax.ShapeDtypeStruct((B, M), jnp.float32),
            jax.ShapeDtypeStruct((B, C), jnp.float32),
        ],
    )(input_encoded, memory_mean)

    BB = 32
    posterior = pl.pallas_call(
        _writer_body,
        grid=(B // BB,),
        in_specs=[
            pl.BlockSpec((M, C), lambda i: (0, 0)),
            pl.BlockSpec((BB, M), lambda i: (i, 0)),
            pl.BlockSpec((BB, C), lambda i: (i, 0)),
        ],
        out_specs=pl.BlockSpec((BB, M, C), lambda i: (i, 0, 0)),
        out_shape=jax.ShapeDtypeStruct((B, M, C), jnp.float32),
    )(memory_mean, oh, delta)

    return z, posterior, kl
